# baseline (device time: 145428 ns/iter reference)
import jax
import jax.numpy as jnp
from jax import lax
from jax.experimental import pallas as pl
from jax.experimental.pallas import tpu as pltpu

N_DEV = 8
B, S, D = 4, 256, 4096
M = B * S
DC = 1024
DCL = DC // N_DEV
H, Dh, Dr = 32, 128, 64
HL = H // N_DEV
KW = HL * Dh
QW = HL * Dr
SL = M // N_DEV
SCALE = (Dh + Dr) ** -0.5

BF = jnp.bfloat16
F32 = jnp.float32

import os
_DIAG_SKIP_AG = bool(int(os.environ.get("DIAG_SKIP_AG", "0")))
_DIAG_SKIP_COMM = bool(int(os.environ.get("DIAG_SKIP_COMM", "0")))
_DIAG_SKIP_AG = _DIAG_SKIP_AG or _DIAG_SKIP_COMM


def _body(x_ref, wdkv_ref, wuk_ref, wuv_ref, wkr_ref, wq_hbm, wqr_hbm, wo_ref,
          out_ref,
          c_all, wukc, wuvc, obuf, orows, wo_stage, wq_st, wqr_st,
          agbuf, outstage,
          a2a_send, a2a_recv, ao_send, ao_recv,
          p1_send, p1_recv, p2_send, p2_recv, p3_send, p3_recv,
          wo_sem, wq_sem, out_sem):
    my = lax.axis_index("i")

    def dot(a, b):
        return lax.dot_general(a, b, (((a.ndim - 1,), (0,)), ((), ())),
                               preferred_element_type=F32)

    def dot_t(a, b):
        return lax.dot_general(a, b, (((1,), (1,)), ((), ())),
                               preferred_element_type=F32)

    DH = D // 2

    def wo_dma(qi):
        jd = lax.rem(my + qi // 2, N_DEV)
        return pltpu.make_async_copy(
            wo_ref.at[pl.ds(jd * KW, KW), pl.ds((qi % 2) * DH, DH)],
            wo_stage.at[qi % 2],
            wo_sem.at[qi % 2],
        )

    started = []

    _pending = [None, None]
    _emitted = [0]

    def emit(cid, val_f32):
        slot = _emitted[0] % 2
        if _pending[slot] is not None:
            _pending[slot].wait()
        outstage[slot, :, :] = val_f32
        dma = pltpu.make_async_copy(
            outstage.at[slot],
            out_ref.at[cid // 2, pl.ds((cid % 2) * SL, SL), :],
            out_sem.at[slot],
        )
        dma.start()
        _pending[slot] = dma
        _emitted[0] += 1

    wq_dma = pltpu.make_async_copy(
        wq_hbm.at[:, pl.ds(my * KW, KW)], wq_st, wq_sem.at[0])
    wqr_dma = pltpu.make_async_copy(
        wqr_hbm.at[:, pl.ds(my * QW, QW)], wqr_st, wq_sem.at[1])
    wq_dma.start()
    wqr_dma.start()
    wo_dma(0).start()
    wo_dma(1).start()

    for d in range(1, 0 if _DIAG_SKIP_COMM else N_DEV):
        p = lax.rem(my + d, N_DEV)
        for t, src, dstbuf in ((0, wuk_ref, wukc), (1, wuv_ref, wuvc)):
            r = pltpu.make_async_remote_copy(
                src_ref=src.at[:, pl.ds(p * KW, KW)],
                dst_ref=dstbuf.at[pl.ds(my * DCL, DCL), :],
                send_sem=a2a_send.at[t, p],
                recv_sem=a2a_recv.at[t, my],
                device_id=(p,),
                device_id_type=pl.DeviceIdType.MESH,
            )
            r.start()
            started.append(r)

    xv = x_ref[:]
    c_all[:, pl.ds(my * DCL, DCL)] = dot(xv, wdkv_ref[:]).astype(BF)
    for d in range(1, 0 if _DIAG_SKIP_COMM else N_DEV):
        p = lax.rem(my + d, N_DEV)
        r = pltpu.make_async_remote_copy(
            src_ref=c_all.at[:, pl.ds(my * DCL, DCL)],
            dst_ref=c_all.at[:, pl.ds(my * DCL, DCL)],
            send_sem=a2a_send.at[2, p],
            recv_sem=a2a_recv.at[2, my],
            device_id=(p,),
            device_id_type=pl.DeviceIdType.MESH,
        )
        r.start()
        started.append(r)

    wukc[pl.ds(my * DCL, DCL), :] = wuk_ref[:, pl.ds(my * KW, KW)]
    wuvc[pl.ds(my * DCL, DCL), :] = wuv_ref[:, pl.ds(my * KW, KW)]

    kr_all = dot(xv, wkr_ref[:]).astype(BF)
    wq_dma.wait()
    q_all = dot(xv, wq_st[:].astype(BF)).astype(BF)
    wqr_dma.wait()
    qr_all = dot(xv, wqr_st[:].astype(BF)).astype(BF)

    for d in range(1, 0 if _DIAG_SKIP_COMM else N_DEV):
        s = lax.rem(my + d, N_DEV)
        for t, dstbuf in ((0, wukc), (1, wuvc)):
            r = pltpu.make_async_remote_copy(
                src_ref=dstbuf.at[pl.ds(s * DCL, DCL), :],
                dst_ref=dstbuf.at[pl.ds(s * DCL, DCL), :],
                send_sem=a2a_send.at[t, s],
                recv_sem=a2a_recv.at[t, s],
                device_id=(s,),
                device_id_type=pl.DeviceIdType.MESH,
            )
            r.wait_recv()
        r = pltpu.make_async_remote_copy(
            src_ref=c_all.at[:, pl.ds(s * DCL, DCL)],
            dst_ref=c_all.at[:, pl.ds(s * DCL, DCL)],
            send_sem=a2a_send.at[2, s],
            recv_sem=a2a_recv.at[2, s],
            device_id=(s,),
            device_id_type=pl.DeviceIdType.MESH,
        )
        r.wait_recv()

    cv = c_all[:]
    k_all = dot(cv, wukc[:]).astype(BF)
    v_all = dot(cv, wuvc[:]).astype(BF)

    for b in range(B):
        r0 = b * S
        krb = kr_all[r0:r0 + S, :]
        for h in range(HL):
            c0 = h * Dh
            q = q_all[r0:r0 + S, c0:c0 + Dh]
            k = k_all[r0:r0 + S, c0:c0 + Dh]
            v = v_all[r0:r0 + S, c0:c0 + Dh]
            qr = qr_all[r0:r0 + S, h * Dr:(h + 1) * Dr]
            sc = (dot_t(q, k) + dot_t(qr, krb)) * SCALE
            mx = jnp.max(sc, axis=1, keepdims=True)
            pr = jnp.exp(sc - mx)
            pr = pr / jnp.sum(pr, axis=1, keepdims=True)
            o = dot(pr.astype(BF), v)
            obuf[r0:r0 + S, c0:c0 + Dh] = o.astype(BF)
        for p in (2 * b, 2 * b + 1) if not _DIAG_SKIP_COMM else ():
            @pl.when(p != my)
            def _send(p=p):
                r = pltpu.make_async_remote_copy(
                    src_ref=obuf.at[pl.ds(p * SL, SL), :],
                    dst_ref=orows.at[pl.ds(my * SL, SL), :],
                    send_sem=ao_send.at[p],
                    recv_sem=ao_recv.at[my],
                    device_id=(p,),
                    device_id_type=pl.DeviceIdType.MESH,
                )
                r.start()

            @pl.when(p == my)
            def _copy(p=p):
                orows[p * SL:(p + 1) * SL, :] = obuf[p * SL:(p + 1) * SL, :]

    acc = [jnp.zeros((SL, DH), F32), jnp.zeros((SL, DH), F32)]
    for d in range(N_DEV):
        jd = lax.rem(my + d, N_DEV)
        if d > 0 and not _DIAG_SKIP_COMM:
            r = pltpu.make_async_remote_copy(
                src_ref=orows.at[pl.ds(jd * SL, SL), :],
                dst_ref=orows.at[pl.ds(jd * SL, SL), :],
                send_sem=ao_send.at[jd],
                recv_sem=ao_recv.at[jd],
                device_id=(jd,),
                device_id_type=pl.DeviceIdType.MESH,
            )
            r.wait_recv()
        ob_j = orows[pl.ds(jd * SL, SL), :]
        for half in (0, 1):
            qi = 2 * d + half
            wo_dma(qi).wait()
            acc[half] = acc[half] + dot(ob_j, wo_stage[half].astype(BF))
            if qi + 2 < 2 * N_DEV:
                wo_dma(qi + 2).start()
    agbuf[pl.ds(my * SL, SL), 0:DH] = acc[0].astype(BF)
    agbuf[pl.ds(my * SL, SL), DH:D] = acc[1].astype(BF)
    emit(my, jnp.concatenate(acc, axis=1))

    if not _DIAG_SKIP_AG:
        nbr = [my ^ 1, my ^ 3, my ^ 4]

        def chunk_ref(cid, half=None):
            if half is None:
                return agbuf.at[pl.ds(cid * SL, SL), :]
            return agbuf.at[pl.ds(cid * SL, SL), pl.ds(half * DH, DH)]

        def ag_rdma(src_cid, dst_cid, send_sem, recv_sem, dev, half=None):
            return pltpu.make_async_remote_copy(
                src_ref=chunk_ref(src_cid, half),
                dst_ref=chunk_ref(dst_cid, half),
                send_sem=send_sem,
                recv_sem=recv_sem,
                device_id=(dev,),
                device_id_type=pl.DeviceIdType.MESH,
            )

        for l in range(3):
            r = ag_rdma(my, my, p1_send.at[l], p1_recv.at[l], nbr[l])
            r.start()
            started.append(r)
        relay_src = [my ^ 3, my ^ 4, my ^ 1]
        need_recv = [1, 2, 0]
        for l in range(3):
            j = need_recv[l]
            ag_rdma(nbr[j], nbr[j], p1_send.at[j], p1_recv.at[j],
                    nbr[j]).wait_recv()
            emit(nbr[j], agbuf[pl.ds(nbr[j] * SL, SL), :].astype(F32))
            r = ag_rdma(relay_src[l], relay_src[l],
                        p2_send.at[l], p2_recv.at[l], nbr[l])
            r.start()
            started.append(r)
        p2_chunk = [my ^ 2, my ^ 7, my ^ 5]
        for l in range(3):
            ag_rdma(p2_chunk[l], p2_chunk[l], p2_send.at[l], p2_recv.at[l],
                    nbr[l]).wait_recv()
            emit(p2_chunk[l], agbuf[pl.ds(p2_chunk[l] * SL, SL), :].astype(F32))
        for l, src_cid in ((0, my ^ 7), (1, my ^ 5)):
            r = ag_rdma(src_cid, src_cid, p3_send.at[l], p3_recv.at[l],
                        nbr[l], half=l)
            r.start()
            started.append(r)
        for l in range(2):
            ag_rdma(my ^ 6, my ^ 6, p3_send.at[l], p3_recv.at[l],
                    nbr[l], half=l).wait_recv()
        emit(my ^ 6, agbuf[pl.ds((my ^ 6) * SL, SL), :].astype(F32))

    for pend in _pending:
        if pend is not None:
            pend.wait()
    for r in started:
        r.wait_send()
    for p in range(0 if _DIAG_SKIP_COMM else N_DEV):
        @pl.when(p != my)
        def _waits(p=p):
            r = pltpu.make_async_remote_copy(
                src_ref=obuf.at[pl.ds(p * SL, SL), :],
                dst_ref=orows.at[pl.ds(my * SL, SL), :],
                send_sem=ao_send.at[p],
                recv_sem=ao_recv.at[my],
                device_id=(p,),
                device_id_type=pl.DeviceIdType.MESH,
            )
            r.wait_send()


def kernel(x, Wdkv, Wuk, Wuv, Wq, Wqr, Wkr, Wo):
    xb = x.reshape(M, D).astype(BF)
    wdkv = Wdkv.astype(BF)
    wuk = Wuk.astype(BF)
    wuv = Wuv.astype(BF)
    wkr = Wkr.astype(BF)

    out = pl.pallas_call(
        _body,
        out_shape=jax.ShapeDtypeStruct((B, S, D), F32),
        in_specs=[pl.BlockSpec(memory_space=pltpu.VMEM)] * 5
        + [pl.BlockSpec(memory_space=pl.ANY)] * 3,
        out_specs=pl.BlockSpec(memory_space=pl.ANY),
        scratch_shapes=[
            pltpu.VMEM((M, DC), BF),
            pltpu.VMEM((DC, KW), BF),
            pltpu.VMEM((DC, KW), BF),
            pltpu.VMEM((M, KW), BF),
            pltpu.VMEM((M, KW), BF),
            pltpu.VMEM((2, KW, D // 2), F32),
            pltpu.VMEM((D, KW), F32),
            pltpu.VMEM((D, QW), F32),
            pltpu.VMEM((M, D), BF),
            pltpu.VMEM((2, SL, D), F32),
            pltpu.SemaphoreType.DMA((3, N_DEV)),
            pltpu.SemaphoreType.DMA((3, N_DEV)),
            pltpu.SemaphoreType.DMA((N_DEV,)),
            pltpu.SemaphoreType.DMA((N_DEV,)),
            pltpu.SemaphoreType.DMA((3,)),
            pltpu.SemaphoreType.DMA((3,)),
            pltpu.SemaphoreType.DMA((3,)),
            pltpu.SemaphoreType.DMA((3,)),
            pltpu.SemaphoreType.DMA((2,)),
            pltpu.SemaphoreType.DMA((2,)),
            pltpu.SemaphoreType.DMA((2,)),
            pltpu.SemaphoreType.DMA((2,)),
            pltpu.SemaphoreType.DMA((2,)),
        ],
        compiler_params=pltpu.CompilerParams(
            vmem_limit_bytes=64 * 1024 * 1024,
        ),
    )(xb, wdkv, wuk, wuv, wkr, Wq, Wqr, Wo)
    return out


# device time: 138078 ns/iter; 1.0532x vs baseline; 1.0532x over previous
import jax
import jax.numpy as jnp
from jax import lax
from jax.experimental import pallas as pl
from jax.experimental.pallas import tpu as pltpu

N_DEV = 8
B, S, D = 4, 256, 4096
M = B * S
DC = 1024
DCL = DC // N_DEV
H, Dh, Dr = 32, 128, 64
HL = H // N_DEV
KW = HL * Dh
QW = HL * Dr
SL = M // N_DEV
SCALE = (Dh + Dr) ** -0.5

BF = jnp.bfloat16
F32 = jnp.float32

import os
_DIAG_SKIP_AG = bool(int(os.environ.get("DIAG_SKIP_AG", "0")))
_DIAG_SKIP_COMM = bool(int(os.environ.get("DIAG_SKIP_COMM", "0")))
_DIAG_SKIP_AG = _DIAG_SKIP_AG or _DIAG_SKIP_COMM


def _body(x_ref, wdkv_ref, wuk_ref, wuv_ref, wkr_ref, wq_hbm, wqr_hbm, wo_ref,
          out_ref,
          c_all, wukc, wuvc, obuf, orows, wo_stage, wq_st, wqr_st,
          agbuf, outstage,
          a2a_send, a2a_recv, ao_send, ao_recv,
          p1_send, p1_recv, p2_send, p2_recv, p3_send, p3_recv,
          wo_sem, wq_sem, out_sem):
    my = lax.axis_index("i")

    def dot(a, b):
        return lax.dot_general(a, b, (((a.ndim - 1,), (0,)), ((), ())),
                               preferred_element_type=F32)

    def dot_t(a, b):
        return lax.dot_general(a, b, (((1,), (1,)), ((), ())),
                               preferred_element_type=F32)

    DH = D // 2

    def wo_dma(seq):
        jd = lax.rem(my + seq % N_DEV, N_DEV)
        return pltpu.make_async_copy(
            wo_ref.at[pl.ds(jd * KW, KW), pl.ds((seq // N_DEV) * DH, DH)],
            wo_stage.at[seq % 2],
            wo_sem.at[seq % 2],
        )

    started = []

    _pending = [None, None]
    _emitted = [0]

    def emit(cid, h, val_f32):
        slot = _emitted[0] % 2
        if _pending[slot] is not None:
            _pending[slot].wait()
        outstage[slot, :, :] = val_f32
        dma = pltpu.make_async_copy(
            outstage.at[slot],
            out_ref.at[cid // 2, pl.ds((cid % 2) * SL, SL), pl.ds(h * DH, DH)],
            out_sem.at[slot],
        )
        dma.start()
        _pending[slot] = dma
        _emitted[0] += 1

    wq_dma = pltpu.make_async_copy(
        wq_hbm.at[:, pl.ds(my * KW, KW)], wq_st, wq_sem.at[0])
    wqr_dma = pltpu.make_async_copy(
        wqr_hbm.at[:, pl.ds(my * QW, QW)], wqr_st, wq_sem.at[1])
    wq_dma.start()
    wqr_dma.start()
    wo_dma(0).start()
    wo_dma(1).start()

    for d in range(1, 0 if _DIAG_SKIP_COMM else N_DEV):
        p = lax.rem(my + d, N_DEV)
        for t, src, dstbuf in ((0, wuk_ref, wukc), (1, wuv_ref, wuvc)):
            r = pltpu.make_async_remote_copy(
                src_ref=src.at[:, pl.ds(p * KW, KW)],
                dst_ref=dstbuf.at[pl.ds(my * DCL, DCL), :],
                send_sem=a2a_send.at[t, p],
                recv_sem=a2a_recv.at[t, my],
                device_id=(p,),
                device_id_type=pl.DeviceIdType.MESH,
            )
            r.start()
            started.append(r)

    xv = x_ref[:]
    c_all[:, pl.ds(my * DCL, DCL)] = dot(xv, wdkv_ref[:]).astype(BF)
    for d in range(1, 0 if _DIAG_SKIP_COMM else N_DEV):
        p = lax.rem(my + d, N_DEV)
        r = pltpu.make_async_remote_copy(
            src_ref=c_all.at[:, pl.ds(my * DCL, DCL)],
            dst_ref=c_all.at[:, pl.ds(my * DCL, DCL)],
            send_sem=a2a_send.at[2, p],
            recv_sem=a2a_recv.at[2, my],
            device_id=(p,),
            device_id_type=pl.DeviceIdType.MESH,
        )
        r.start()
        started.append(r)

    wukc[pl.ds(my * DCL, DCL), :] = wuk_ref[:, pl.ds(my * KW, KW)]
    wuvc[pl.ds(my * DCL, DCL), :] = wuv_ref[:, pl.ds(my * KW, KW)]

    kr_all = dot(xv, wkr_ref[:]).astype(BF)
    wq_dma.wait()
    q_all = dot(xv, wq_st[:].astype(BF)).astype(BF)
    wqr_dma.wait()
    qr_all = dot(xv, wqr_st[:].astype(BF)).astype(BF)

    for d in range(1, 0 if _DIAG_SKIP_COMM else N_DEV):
        s = lax.rem(my + d, N_DEV)
        for t, dstbuf in ((0, wukc), (1, wuvc)):
            r = pltpu.make_async_remote_copy(
                src_ref=dstbuf.at[pl.ds(s * DCL, DCL), :],
                dst_ref=dstbuf.at[pl.ds(s * DCL, DCL), :],
                send_sem=a2a_send.at[t, s],
                recv_sem=a2a_recv.at[t, s],
                device_id=(s,),
                device_id_type=pl.DeviceIdType.MESH,
            )
            r.wait_recv()
        r = pltpu.make_async_remote_copy(
            src_ref=c_all.at[:, pl.ds(s * DCL, DCL)],
            dst_ref=c_all.at[:, pl.ds(s * DCL, DCL)],
            send_sem=a2a_send.at[2, s],
            recv_sem=a2a_recv.at[2, s],
            device_id=(s,),
            device_id_type=pl.DeviceIdType.MESH,
        )
        r.wait_recv()

    cv = c_all[:]
    k_all = dot(cv, wukc[:]).astype(BF)
    v_all = dot(cv, wuvc[:]).astype(BF)

    for b in range(B):
        r0 = b * S
        krb = kr_all[r0:r0 + S, :]
        for h in range(HL):
            c0 = h * Dh
            q = q_all[r0:r0 + S, c0:c0 + Dh]
            k = k_all[r0:r0 + S, c0:c0 + Dh]
            v = v_all[r0:r0 + S, c0:c0 + Dh]
            qr = qr_all[r0:r0 + S, h * Dr:(h + 1) * Dr]
            sc = (dot_t(q, k) + dot_t(qr, krb)) * SCALE
            mx = jnp.max(sc, axis=1, keepdims=True)
            pr = jnp.exp(sc - mx)
            pr = pr / jnp.sum(pr, axis=1, keepdims=True)
            o = dot(pr.astype(BF), v)
            obuf[r0:r0 + S, c0:c0 + Dh] = o.astype(BF)
        for p in (2 * b, 2 * b + 1) if not _DIAG_SKIP_COMM else ():
            @pl.when(p != my)
            def _send(p=p):
                r = pltpu.make_async_remote_copy(
                    src_ref=obuf.at[pl.ds(p * SL, SL), :],
                    dst_ref=orows.at[pl.ds(my * SL, SL), :],
                    send_sem=ao_send.at[p],
                    recv_sem=ao_recv.at[my],
                    device_id=(p,),
                    device_id_type=pl.DeviceIdType.MESH,
                )
                r.start()

            @pl.when(p == my)
            def _copy(p=p):
                orows[p * SL:(p + 1) * SL, :] = obuf[p * SL:(p + 1) * SL, :]

    nbr = [my ^ 1, my ^ 3, my ^ 4]
    relay_src = [my ^ 3, my ^ 4, my ^ 1]
    need_recv = [1, 2, 0]
    p2_chunk = [my ^ 2, my ^ 7, my ^ 5]

    def chunk_val(cid, h):
        return agbuf[pl.ds(cid * SL, SL), pl.ds(h * DH, DH)]

    def ag_rdma(cid, send_sem, recv_sem, dev, h):
        ref = agbuf.at[pl.ds(cid * SL, SL), pl.ds(h * DH, DH)]
        return pltpu.make_async_remote_copy(
            src_ref=ref, dst_ref=ref, send_sem=send_sem, recv_sem=recv_sem,
            device_id=(dev,), device_id_type=pl.DeviceIdType.MESH,
        )

    for h in (0, 1):
        acc = jnp.zeros((SL, DH), F32)
        for d in range(N_DEV):
            jd = lax.rem(my + d, N_DEV)
            if h == 0 and d > 0 and not _DIAG_SKIP_COMM:
                r = pltpu.make_async_remote_copy(
                    src_ref=orows.at[pl.ds(jd * SL, SL), :],
                    dst_ref=orows.at[pl.ds(jd * SL, SL), :],
                    send_sem=ao_send.at[jd],
                    recv_sem=ao_recv.at[jd],
                    device_id=(jd,),
                    device_id_type=pl.DeviceIdType.MESH,
                )
                r.wait_recv()
            seq = h * N_DEV + d
            wo_dma(seq).wait()
            acc = acc + dot(orows[pl.ds(jd * SL, SL), :],
                            wo_stage[seq % 2].astype(BF))
            if seq + 2 < 2 * N_DEV:
                wo_dma(seq + 2).start()
        agbuf[pl.ds(my * SL, SL), pl.ds(h * DH, DH)] = acc.astype(BF)
        emit(my, h, acc)
        if not _DIAG_SKIP_AG:
            for l in range(3):
                r = ag_rdma(my, p1_send.at[l, h], p1_recv.at[l, h], nbr[l], h)
                r.start()
                started.append(r)

    if not _DIAG_SKIP_AG:
        for h in (0, 1):
            for l in range(3):
                j = need_recv[l]
                ag_rdma(nbr[j], p1_send.at[j, h], p1_recv.at[j, h],
                        nbr[j], h).wait_recv()
                emit(nbr[j], h, chunk_val(nbr[j], h).astype(F32))
                r = ag_rdma(relay_src[l], p2_send.at[l, h], p2_recv.at[l, h],
                            nbr[l], h)
                r.start()
                started.append(r)
        for h in (0, 1):
            for l in range(3):
                ag_rdma(p2_chunk[l], p2_send.at[l, h], p2_recv.at[l, h],
                        nbr[l], h).wait_recv()
                emit(p2_chunk[l], h, chunk_val(p2_chunk[l], h).astype(F32))
            src_cid = my ^ (7 if h == 0 else 5)
            r = ag_rdma(src_cid, p3_send.at[h], p3_recv.at[h], nbr[h], h)
            r.start()
            started.append(r)
        for h in (0, 1):
            ag_rdma(my ^ 6, p3_send.at[h], p3_recv.at[h], nbr[h], h).wait_recv()
            emit(my ^ 6, h, chunk_val(my ^ 6, h).astype(F32))

    for pend in _pending:
        if pend is not None:
            pend.wait()
    for r in started:
        r.wait_send()
    for p in range(0 if _DIAG_SKIP_COMM else N_DEV):
        @pl.when(p != my)
        def _waits(p=p):
            r = pltpu.make_async_remote_copy(
                src_ref=obuf.at[pl.ds(p * SL, SL), :],
                dst_ref=orows.at[pl.ds(my * SL, SL), :],
                send_sem=ao_send.at[p],
                recv_sem=ao_recv.at[my],
                device_id=(p,),
                device_id_type=pl.DeviceIdType.MESH,
            )
            r.wait_send()


def kernel(x, Wdkv, Wuk, Wuv, Wq, Wqr, Wkr, Wo):
    xb = x.reshape(M, D).astype(BF)
    wdkv = Wdkv.astype(BF)
    wuk = Wuk.astype(BF)
    wuv = Wuv.astype(BF)
    wkr = Wkr.astype(BF)

    out = pl.pallas_call(
        _body,
        out_shape=jax.ShapeDtypeStruct((B, S, D), F32),
        in_specs=[pl.BlockSpec(memory_space=pltpu.VMEM)] * 5
        + [pl.BlockSpec(memory_space=pl.ANY)] * 3,
        out_specs=pl.BlockSpec(memory_space=pl.ANY),
        scratch_shapes=[
            pltpu.VMEM((M, DC), BF),
            pltpu.VMEM((DC, KW), BF),
            pltpu.VMEM((DC, KW), BF),
            pltpu.VMEM((M, KW), BF),
            pltpu.VMEM((M, KW), BF),
            pltpu.VMEM((2, KW, D // 2), F32),
            pltpu.VMEM((D, KW), F32),
            pltpu.VMEM((D, QW), F32),
            pltpu.VMEM((M, D), BF),
            pltpu.VMEM((2, SL, D // 2), F32),
            pltpu.SemaphoreType.DMA((3, N_DEV)),
            pltpu.SemaphoreType.DMA((3, N_DEV)),
            pltpu.SemaphoreType.DMA((N_DEV,)),
            pltpu.SemaphoreType.DMA((N_DEV,)),
            pltpu.SemaphoreType.DMA((3, 2)),
            pltpu.SemaphoreType.DMA((3, 2)),
            pltpu.SemaphoreType.DMA((3, 2)),
            pltpu.SemaphoreType.DMA((3, 2)),
            pltpu.SemaphoreType.DMA((2,)),
            pltpu.SemaphoreType.DMA((2,)),
            pltpu.SemaphoreType.DMA((2,)),
            pltpu.SemaphoreType.DMA((2,)),
            pltpu.SemaphoreType.DMA((2,)),
        ],
        compiler_params=pltpu.CompilerParams(
            vmem_limit_bytes=64 * 1024 * 1024,
        ),
    )(xb, wdkv, wuk, wuv, wkr, Wq, Wqr, Wo)
    return out


# device time: 137954 ns/iter; 1.0542x vs baseline; 1.0009x over previous
import jax
import jax.numpy as jnp
from jax import lax
from jax.experimental import pallas as pl
from jax.experimental.pallas import tpu as pltpu

N_DEV = 8
B, S, D = 4, 256, 4096
M = B * S
DC = 1024
DCL = DC // N_DEV
H, Dh, Dr = 32, 128, 64
HL = H // N_DEV
KW = HL * Dh
QW = HL * Dr
SL = M // N_DEV
SCALE = (Dh + Dr) ** -0.5

BF = jnp.bfloat16
F32 = jnp.float32

import os
_DIAG_SKIP_AG = bool(int(os.environ.get("DIAG_SKIP_AG", "0")))
_DIAG_SKIP_COMM = bool(int(os.environ.get("DIAG_SKIP_COMM", "0")))
_DIAG_SKIP_AG = _DIAG_SKIP_AG or _DIAG_SKIP_COMM


def _body(x_ref, wdkv_ref, wuk_ref, wuv_ref, wkr_ref, wq_hbm, wqr_hbm, wo_ref,
          out_ref,
          c_all, wukc, wuvc, obuf, orows, wo_stage, wq_st, wqr_st,
          a2a_send, a2a_recv, ao_send, ao_recv,
          p1_send, p1_recv, p2_send, p2_recv, p3_send, p3_recv,
          wo_sem, wq_sem):
    my = lax.axis_index("i")

    def dot(a, b):
        return lax.dot_general(a, b, (((a.ndim - 1,), (0,)), ((), ())),
                               preferred_element_type=F32)

    def dot_t(a, b):
        return lax.dot_general(a, b, (((1,), (1,)), ((), ())),
                               preferred_element_type=F32)

    DH = D // 2

    def wo_dma(seq):
        jd = lax.rem(my + seq % N_DEV, N_DEV)
        return pltpu.make_async_copy(
            wo_ref.at[pl.ds(jd * KW, KW), pl.ds((seq // N_DEV) * DH, DH)],
            wo_stage.at[seq % 2],
            wo_sem.at[seq % 2],
        )

    started = []

    wq_dma = pltpu.make_async_copy(
        wq_hbm.at[:, pl.ds(my * KW, KW)], wq_st, wq_sem.at[0])
    wqr_dma = pltpu.make_async_copy(
        wqr_hbm.at[:, pl.ds(my * QW, QW)], wqr_st, wq_sem.at[1])
    wq_dma.start()
    wqr_dma.start()
    wo_dma(0).start()
    wo_dma(1).start()

    for d in range(1, 0 if _DIAG_SKIP_COMM else N_DEV):
        p = lax.rem(my + d, N_DEV)
        for t, src, dstbuf in ((0, wuk_ref, wukc), (1, wuv_ref, wuvc)):
            r = pltpu.make_async_remote_copy(
                src_ref=src.at[:, pl.ds(p * KW, KW)],
                dst_ref=dstbuf.at[pl.ds(my * DCL, DCL), :],
                send_sem=a2a_send.at[t, p],
                recv_sem=a2a_recv.at[t, my],
                device_id=(p,),
                device_id_type=pl.DeviceIdType.MESH,
            )
            r.start()
            started.append(r)

    xv = x_ref[:]
    c_all[:, pl.ds(my * DCL, DCL)] = dot(xv, wdkv_ref[:]).astype(BF)
    for d in range(1, 0 if _DIAG_SKIP_COMM else N_DEV):
        p = lax.rem(my + d, N_DEV)
        r = pltpu.make_async_remote_copy(
            src_ref=c_all.at[:, pl.ds(my * DCL, DCL)],
            dst_ref=c_all.at[:, pl.ds(my * DCL, DCL)],
            send_sem=a2a_send.at[2, p],
            recv_sem=a2a_recv.at[2, my],
            device_id=(p,),
            device_id_type=pl.DeviceIdType.MESH,
        )
        r.start()
        started.append(r)

    wukc[pl.ds(my * DCL, DCL), :] = wuk_ref[:, pl.ds(my * KW, KW)]
    wuvc[pl.ds(my * DCL, DCL), :] = wuv_ref[:, pl.ds(my * KW, KW)]

    kr_all = dot(xv, wkr_ref[:]).astype(BF)
    wq_dma.wait()
    q_all = dot(xv, wq_st[:].astype(BF)).astype(BF)
    wqr_dma.wait()
    qr_all = dot(xv, wqr_st[:].astype(BF)).astype(BF)

    for d in range(1, 0 if _DIAG_SKIP_COMM else N_DEV):
        s = lax.rem(my + d, N_DEV)
        for t, dstbuf in ((0, wukc), (1, wuvc)):
            r = pltpu.make_async_remote_copy(
                src_ref=dstbuf.at[pl.ds(s * DCL, DCL), :],
                dst_ref=dstbuf.at[pl.ds(s * DCL, DCL), :],
                send_sem=a2a_send.at[t, s],
                recv_sem=a2a_recv.at[t, s],
                device_id=(s,),
                device_id_type=pl.DeviceIdType.MESH,
            )
            r.wait_recv()
        r = pltpu.make_async_remote_copy(
            src_ref=c_all.at[:, pl.ds(s * DCL, DCL)],
            dst_ref=c_all.at[:, pl.ds(s * DCL, DCL)],
            send_sem=a2a_send.at[2, s],
            recv_sem=a2a_recv.at[2, s],
            device_id=(s,),
            device_id_type=pl.DeviceIdType.MESH,
        )
        r.wait_recv()

    cv = c_all[:]
    k_all = dot(cv, wukc[:]).astype(BF)
    v_all = dot(cv, wuvc[:]).astype(BF)

    for b in range(B):
        r0 = b * S
        krb = kr_all[r0:r0 + S, :]
        for h in range(HL):
            c0 = h * Dh
            q = q_all[r0:r0 + S, c0:c0 + Dh]
            k = k_all[r0:r0 + S, c0:c0 + Dh]
            v = v_all[r0:r0 + S, c0:c0 + Dh]
            qr = qr_all[r0:r0 + S, h * Dr:(h + 1) * Dr]
            sc = (dot_t(q, k) + dot_t(qr, krb)) * SCALE
            mx = jnp.max(sc, axis=1, keepdims=True)
            pr = jnp.exp(sc - mx)
            pr = pr / jnp.sum(pr, axis=1, keepdims=True)
            o = dot(pr.astype(BF), v)
            obuf[r0:r0 + S, c0:c0 + Dh] = o.astype(BF)
        for p in (2 * b, 2 * b + 1) if not _DIAG_SKIP_COMM else ():
            @pl.when(p != my)
            def _send(p=p):
                r = pltpu.make_async_remote_copy(
                    src_ref=obuf.at[pl.ds(p * SL, SL), :],
                    dst_ref=orows.at[pl.ds(my * SL, SL), :],
                    send_sem=ao_send.at[p],
                    recv_sem=ao_recv.at[my],
                    device_id=(p,),
                    device_id_type=pl.DeviceIdType.MESH,
                )
                r.start()

            @pl.when(p == my)
            def _copy(p=p):
                orows[p * SL:(p + 1) * SL, :] = obuf[p * SL:(p + 1) * SL, :]

    nbr = [my ^ 1, my ^ 3, my ^ 4]
    relay_src = [my ^ 3, my ^ 4, my ^ 1]
    need_recv = [1, 2, 0]
    p2_chunk = [my ^ 2, my ^ 7, my ^ 5]

    def ag_rdma(cid, send_sem, recv_sem, dev, h):
        ref = out_ref.at[pl.ds(cid * SL, SL), pl.ds(h * DH, DH)]
        return pltpu.make_async_remote_copy(
            src_ref=ref, dst_ref=ref, send_sem=send_sem, recv_sem=recv_sem,
            device_id=(dev,), device_id_type=pl.DeviceIdType.MESH,
        )

    for h in (0, 1):
        acc = jnp.zeros((SL, DH), F32)
        for d in range(N_DEV):
            jd = lax.rem(my + d, N_DEV)
            if h == 0 and d > 0 and not _DIAG_SKIP_COMM:
                r = pltpu.make_async_remote_copy(
                    src_ref=orows.at[pl.ds(jd * SL, SL), :],
                    dst_ref=orows.at[pl.ds(jd * SL, SL), :],
                    send_sem=ao_send.at[jd],
                    recv_sem=ao_recv.at[jd],
                    device_id=(jd,),
                    device_id_type=pl.DeviceIdType.MESH,
                )
                r.wait_recv()
            seq = h * N_DEV + d
            wo_dma(seq).wait()
            acc = acc + dot(orows[pl.ds(jd * SL, SL), :],
                            wo_stage[seq % 2].astype(BF))
            if seq + 2 < 2 * N_DEV:
                wo_dma(seq + 2).start()
        out_ref[pl.ds(my * SL, SL), pl.ds(h * DH, DH)] = acc.astype(BF)
        if not _DIAG_SKIP_AG:
            for l in range(3):
                r = ag_rdma(my, p1_send.at[l, h], p1_recv.at[l, h], nbr[l], h)
                r.start()
                started.append(r)

    if not _DIAG_SKIP_AG:
        for h in (0, 1):
            for l in range(3):
                j = need_recv[l]
                ag_rdma(nbr[j], p1_send.at[j, h], p1_recv.at[j, h],
                        nbr[j], h).wait_recv()
                r = ag_rdma(relay_src[l], p2_send.at[l, h], p2_recv.at[l, h],
                            nbr[l], h)
                r.start()
                started.append(r)
        for h in (0, 1):
            for l in range(3):
                ag_rdma(p2_chunk[l], p2_send.at[l, h], p2_recv.at[l, h],
                        nbr[l], h).wait_recv()
            src_cid = my ^ (7 if h == 0 else 5)
            r = ag_rdma(src_cid, p3_send.at[h], p3_recv.at[h], nbr[h], h)
            r.start()
            started.append(r)
        for h in (0, 1):
            ag_rdma(my ^ 6, p3_send.at[h], p3_recv.at[h], nbr[h], h).wait_recv()

    for r in started:
        r.wait_send()
    for p in range(0 if _DIAG_SKIP_COMM else N_DEV):
        @pl.when(p != my)
        def _waits(p=p):
            r = pltpu.make_async_remote_copy(
                src_ref=obuf.at[pl.ds(p * SL, SL), :],
                dst_ref=orows.at[pl.ds(my * SL, SL), :],
                send_sem=ao_send.at[p],
                recv_sem=ao_recv.at[my],
                device_id=(p,),
                device_id_type=pl.DeviceIdType.MESH,
            )
            r.wait_send()


def kernel(x, Wdkv, Wuk, Wuv, Wq, Wqr, Wkr, Wo):
    xb = x.reshape(M, D).astype(BF)
    wdkv = Wdkv.astype(BF)
    wuk = Wuk.astype(BF)
    wuv = Wuv.astype(BF)
    wkr = Wkr.astype(BF)

    out = pl.pallas_call(
        _body,
        out_shape=jax.ShapeDtypeStruct((M, D), BF),
        in_specs=[pl.BlockSpec(memory_space=pltpu.VMEM)] * 5
        + [pl.BlockSpec(memory_space=pl.ANY)] * 3,
        out_specs=pl.BlockSpec(memory_space=pltpu.VMEM),
        scratch_shapes=[
            pltpu.VMEM((M, DC), BF),
            pltpu.VMEM((DC, KW), BF),
            pltpu.VMEM((DC, KW), BF),
            pltpu.VMEM((M, KW), BF),
            pltpu.VMEM((M, KW), BF),
            pltpu.VMEM((2, KW, D // 2), F32),
            pltpu.VMEM((D, KW), F32),
            pltpu.VMEM((D, QW), F32),
            pltpu.SemaphoreType.DMA((3, N_DEV)),
            pltpu.SemaphoreType.DMA((3, N_DEV)),
            pltpu.SemaphoreType.DMA((N_DEV,)),
            pltpu.SemaphoreType.DMA((N_DEV,)),
            pltpu.SemaphoreType.DMA((3, 2)),
            pltpu.SemaphoreType.DMA((3, 2)),
            pltpu.SemaphoreType.DMA((3, 2)),
            pltpu.SemaphoreType.DMA((3, 2)),
            pltpu.SemaphoreType.DMA((2,)),
            pltpu.SemaphoreType.DMA((2,)),
            pltpu.SemaphoreType.DMA((2,)),
            pltpu.SemaphoreType.DMA((2,)),
        ],
        compiler_params=pltpu.CompilerParams(
            vmem_limit_bytes=64 * 1024 * 1024,
        ),
    )(xb, wdkv, wuk, wuv, wkr, Wq, Wqr, Wo)
    return out.reshape(B, S, D).astype(jnp.float32)


# device time: 131311 ns/iter; 1.1075x vs baseline; 1.0506x over previous
import jax
import jax.numpy as jnp
from jax import lax
from jax.experimental import pallas as pl
from jax.experimental.pallas import tpu as pltpu

N_DEV = 8
B, S, D = 4, 256, 4096
M = B * S
DC = 1024
DCL = DC // N_DEV
H, Dh, Dr = 32, 128, 64
HL = H // N_DEV
KW = HL * Dh
QW = HL * Dr
SL = M // N_DEV
SCALE = (Dh + Dr) ** -0.5

BF = jnp.bfloat16
F32 = jnp.float32

import os
_DIAG_SKIP_AG = bool(int(os.environ.get("DIAG_SKIP_AG", "0")))
_DIAG_SKIP_COMM = bool(int(os.environ.get("DIAG_SKIP_COMM", "0")))
_DIAG_SKIP_AG = _DIAG_SKIP_AG or _DIAG_SKIP_COMM


def _body(x_hbm, wdkv_ref, wuk_ref, wuv_ref, wkr_ref, wq_hbm, wqr_hbm, wo_ref,
          out_ref,
          c_all, wukc, wuvc, obuf, orows, wo_stage, wq_st, wqr_st, xb_buf, xst,
          a2a_send, a2a_recv, ao_send, ao_recv,
          p1_send, p1_recv, p2_send, p2_recv, p3_send, p3_recv,
          wo_sem, wq_sem, x_sem):
    my = lax.axis_index("i")

    def dot(a, b):
        return lax.dot_general(a, b, (((a.ndim - 1,), (0,)), ((), ())),
                               preferred_element_type=F32)

    def dot_t(a, b):
        return lax.dot_general(a, b, (((1,), (1,)), ((), ())),
                               preferred_element_type=F32)

    DH = D // 2

    def wo_dma(seq):
        jd = lax.rem(my + seq % N_DEV, N_DEV)
        return pltpu.make_async_copy(
            wo_ref.at[pl.ds(jd * KW, KW), pl.ds((seq // N_DEV) * DH, DH)],
            wo_stage.at[seq % 2],
            wo_sem.at[seq % 2],
        )

    started = []

    wq_dma = pltpu.make_async_copy(
        wq_hbm.at[:, pl.ds(my * KW, KW)], wq_st, wq_sem.at[0])
    wqr_dma = pltpu.make_async_copy(
        wqr_hbm.at[:, pl.ds(my * QW, QW)], wqr_st, wq_sem.at[1])
    def x_dma(t):
        return pltpu.make_async_copy(
            x_hbm.at[pl.ds(t * S, S), :], xst.at[t % 2], x_sem.at[t % 2])

    wq_dma.start()
    wqr_dma.start()
    wo_dma(0).start()
    wo_dma(1).start()
    x_dma(0).start()
    x_dma(1).start()

    for d in range(1, 0 if _DIAG_SKIP_COMM else N_DEV):
        p = lax.rem(my + d, N_DEV)
        for t, src, dstbuf in ((0, wuk_ref, wukc), (1, wuv_ref, wuvc)):
            r = pltpu.make_async_remote_copy(
                src_ref=src.at[:, pl.ds(p * KW, KW)],
                dst_ref=dstbuf.at[pl.ds(my * DCL, DCL), :],
                send_sem=a2a_send.at[t, p],
                recv_sem=a2a_recv.at[t, my],
                device_id=(p,),
                device_id_type=pl.DeviceIdType.MESH,
            )
            r.start()
            started.append(r)

    for t in range(4):
        x_dma(t).wait()
        xb_buf[t * S:(t + 1) * S, :] = xst[t % 2].astype(BF)
        if t + 2 < 4:
            x_dma(t + 2).start()

    xv = xb_buf[:]
    c_all[:, pl.ds(my * DCL, DCL)] = dot(xv, wdkv_ref[:]).astype(BF)
    for d in range(1, 0 if _DIAG_SKIP_COMM else N_DEV):
        p = lax.rem(my + d, N_DEV)
        r = pltpu.make_async_remote_copy(
            src_ref=c_all.at[:, pl.ds(my * DCL, DCL)],
            dst_ref=c_all.at[:, pl.ds(my * DCL, DCL)],
            send_sem=a2a_send.at[2, p],
            recv_sem=a2a_recv.at[2, my],
            device_id=(p,),
            device_id_type=pl.DeviceIdType.MESH,
        )
        r.start()
        started.append(r)

    wukc[pl.ds(my * DCL, DCL), :] = wuk_ref[:, pl.ds(my * KW, KW)]
    wuvc[pl.ds(my * DCL, DCL), :] = wuv_ref[:, pl.ds(my * KW, KW)]

    kr_all = dot(xv, wkr_ref[:]).astype(BF)
    wq_dma.wait()
    q_all = dot(xv, wq_st[:].astype(BF)).astype(BF)
    wqr_dma.wait()
    qr_all = dot(xv, wqr_st[:].astype(BF)).astype(BF)

    for d in range(1, 0 if _DIAG_SKIP_COMM else N_DEV):
        s = lax.rem(my + d, N_DEV)
        for t, dstbuf in ((0, wukc), (1, wuvc)):
            r = pltpu.make_async_remote_copy(
                src_ref=dstbuf.at[pl.ds(s * DCL, DCL), :],
                dst_ref=dstbuf.at[pl.ds(s * DCL, DCL), :],
                send_sem=a2a_send.at[t, s],
                recv_sem=a2a_recv.at[t, s],
                device_id=(s,),
                device_id_type=pl.DeviceIdType.MESH,
            )
            r.wait_recv()
        r = pltpu.make_async_remote_copy(
            src_ref=c_all.at[:, pl.ds(s * DCL, DCL)],
            dst_ref=c_all.at[:, pl.ds(s * DCL, DCL)],
            send_sem=a2a_send.at[2, s],
            recv_sem=a2a_recv.at[2, s],
            device_id=(s,),
            device_id_type=pl.DeviceIdType.MESH,
        )
        r.wait_recv()

    cv = c_all[:]
    k_all = dot(cv, wukc[:]).astype(BF)
    v_all = dot(cv, wuvc[:]).astype(BF)

    for b in range(B):
        r0 = b * S
        krb = kr_all[r0:r0 + S, :]
        for h in range(HL):
            c0 = h * Dh
            q = q_all[r0:r0 + S, c0:c0 + Dh]
            k = k_all[r0:r0 + S, c0:c0 + Dh]
            v = v_all[r0:r0 + S, c0:c0 + Dh]
            qr = qr_all[r0:r0 + S, h * Dr:(h + 1) * Dr]
            sc = (dot_t(q, k) + dot_t(qr, krb)) * SCALE
            mx = jnp.max(sc, axis=1, keepdims=True)
            pr = jnp.exp(sc - mx)
            pr = pr / jnp.sum(pr, axis=1, keepdims=True)
            o = dot(pr.astype(BF), v)
            obuf[r0:r0 + S, c0:c0 + Dh] = o.astype(BF)
        for p in (2 * b, 2 * b + 1) if not _DIAG_SKIP_COMM else ():
            @pl.when(p != my)
            def _send(p=p):
                r = pltpu.make_async_remote_copy(
                    src_ref=obuf.at[pl.ds(p * SL, SL), :],
                    dst_ref=orows.at[pl.ds(my * SL, SL), :],
                    send_sem=ao_send.at[p],
                    recv_sem=ao_recv.at[my],
                    device_id=(p,),
                    device_id_type=pl.DeviceIdType.MESH,
                )
                r.start()

            @pl.when(p == my)
            def _copy(p=p):
                orows[p * SL:(p + 1) * SL, :] = obuf[p * SL:(p + 1) * SL, :]

    nbr = [my ^ 1, my ^ 3, my ^ 4]
    relay_src = [my ^ 3, my ^ 4, my ^ 1]
    need_recv = [1, 2, 0]
    p2_chunk = [my ^ 2, my ^ 7, my ^ 5]

    def ag_rdma(cid, send_sem, recv_sem, dev, h):
        ref = out_ref.at[pl.ds(cid * SL, SL), pl.ds(h * DH, DH)]
        return pltpu.make_async_remote_copy(
            src_ref=ref, dst_ref=ref, send_sem=send_sem, recv_sem=recv_sem,
            device_id=(dev,), device_id_type=pl.DeviceIdType.MESH,
        )

    for h in (0, 1):
        acc = jnp.zeros((SL, DH), F32)
        for d in range(N_DEV):
            jd = lax.rem(my + d, N_DEV)
            if h == 0 and d > 0 and not _DIAG_SKIP_COMM:
                r = pltpu.make_async_remote_copy(
                    src_ref=orows.at[pl.ds(jd * SL, SL), :],
                    dst_ref=orows.at[pl.ds(jd * SL, SL), :],
                    send_sem=ao_send.at[jd],
                    recv_sem=ao_recv.at[jd],
                    device_id=(jd,),
                    device_id_type=pl.DeviceIdType.MESH,
                )
                r.wait_recv()
            seq = h * N_DEV + d
            wo_dma(seq).wait()
            acc = acc + dot(orows[pl.ds(jd * SL, SL), :],
                            wo_stage[seq % 2].astype(BF))
            if seq + 2 < 2 * N_DEV:
                wo_dma(seq + 2).start()
        out_ref[pl.ds(my * SL, SL), pl.ds(h * DH, DH)] = acc.astype(BF)
        if not _DIAG_SKIP_AG:
            for l in range(3):
                r = ag_rdma(my, p1_send.at[l, h], p1_recv.at[l, h], nbr[l], h)
                r.start()
                started.append(r)

    if not _DIAG_SKIP_AG:
        for h in (0, 1):
            for l in range(3):
                j = need_recv[l]
                ag_rdma(nbr[j], p1_send.at[j, h], p1_recv.at[j, h],
                        nbr[j], h).wait_recv()
                r = ag_rdma(relay_src[l], p2_send.at[l, h], p2_recv.at[l, h],
                            nbr[l], h)
                r.start()
                started.append(r)
        for h in (0, 1):
            for l in range(3):
                ag_rdma(p2_chunk[l], p2_send.at[l, h], p2_recv.at[l, h],
                        nbr[l], h).wait_recv()
            src_cid = my ^ (7 if h == 0 else 5)
            r = ag_rdma(src_cid, p3_send.at[h], p3_recv.at[h], nbr[h], h)
            r.start()
            started.append(r)
        for h in (0, 1):
            ag_rdma(my ^ 6, p3_send.at[h], p3_recv.at[h], nbr[h], h).wait_recv()

    for r in started:
        r.wait_send()
    for p in range(0 if _DIAG_SKIP_COMM else N_DEV):
        @pl.when(p != my)
        def _waits(p=p):
            r = pltpu.make_async_remote_copy(
                src_ref=obuf.at[pl.ds(p * SL, SL), :],
                dst_ref=orows.at[pl.ds(my * SL, SL), :],
                send_sem=ao_send.at[p],
                recv_sem=ao_recv.at[my],
                device_id=(p,),
                device_id_type=pl.DeviceIdType.MESH,
            )
            r.wait_send()


def kernel(x, Wdkv, Wuk, Wuv, Wq, Wqr, Wkr, Wo):
    xf = x.reshape(M, D)
    wdkv = Wdkv.astype(BF)
    wuk = Wuk.astype(BF)
    wuv = Wuv.astype(BF)
    wkr = Wkr.astype(BF)

    out = pl.pallas_call(
        _body,
        out_shape=jax.ShapeDtypeStruct((M, D), BF),
        in_specs=[pl.BlockSpec(memory_space=pl.ANY)]
        + [pl.BlockSpec(memory_space=pltpu.VMEM)] * 4
        + [pl.BlockSpec(memory_space=pl.ANY)] * 3,
        out_specs=pl.BlockSpec(memory_space=pltpu.VMEM),
        scratch_shapes=[
            pltpu.VMEM((M, DC), BF),
            pltpu.VMEM((DC, KW), BF),
            pltpu.VMEM((DC, KW), BF),
            pltpu.VMEM((M, KW), BF),
            pltpu.VMEM((M, KW), BF),
            pltpu.VMEM((2, KW, D // 2), F32),
            pltpu.VMEM((D, KW), F32),
            pltpu.VMEM((D, QW), F32),
            pltpu.VMEM((M, D), BF),
            pltpu.VMEM((2, S, D), F32),
            pltpu.SemaphoreType.DMA((3, N_DEV)),
            pltpu.SemaphoreType.DMA((3, N_DEV)),
            pltpu.SemaphoreType.DMA((N_DEV,)),
            pltpu.SemaphoreType.DMA((N_DEV,)),
            pltpu.SemaphoreType.DMA((3, 2)),
            pltpu.SemaphoreType.DMA((3, 2)),
            pltpu.SemaphoreType.DMA((3, 2)),
            pltpu.SemaphoreType.DMA((3, 2)),
            pltpu.SemaphoreType.DMA((2,)),
            pltpu.SemaphoreType.DMA((2,)),
            pltpu.SemaphoreType.DMA((2,)),
            pltpu.SemaphoreType.DMA((2,)),
            pltpu.SemaphoreType.DMA((2,)),
        ],
        compiler_params=pltpu.CompilerParams(
            vmem_limit_bytes=64 * 1024 * 1024,
        ),
    )(xf, wdkv, wuk, wuv, wkr, Wq, Wqr, Wo)
    return out.reshape(B, S, D).astype(jnp.float32)


# device time: 124359 ns/iter; 1.1694x vs baseline; 1.0559x over previous
import jax
import jax.numpy as jnp
from jax import lax
from jax.experimental import pallas as pl
from jax.experimental.pallas import tpu as pltpu

N_DEV = 8
B, S, D = 4, 256, 4096
M = B * S
DC = 1024
DCL = DC // N_DEV
H, Dh, Dr = 32, 128, 64
HL = H // N_DEV
KW = HL * Dh
QW = HL * Dr
SL = M // N_DEV
SCALE = (Dh + Dr) ** -0.5

BF = jnp.bfloat16
F32 = jnp.float32

import os
_DIAG_SKIP_AG = bool(int(os.environ.get("DIAG_SKIP_AG", "0")))
_DIAG_SKIP_COMM = bool(int(os.environ.get("DIAG_SKIP_COMM", "0")))
_DIAG_SKIP_AG = _DIAG_SKIP_AG or _DIAG_SKIP_COMM


def _body(x_hbm, wdkv_hbm, wuk_hbm, wuv_hbm, wkr_ref, wq_hbm, wqr_hbm, wo_ref,
          out_ref,
          c_all, wukc, wuvc, obuf, orows, wo_stage, wq_st, wqr_st, xb_buf, xst,
          wdkv_f, wuk_f, wuv_f, wdkv_ref, wuk_ref, wuv_ref,
          a2a_send, a2a_recv, ao_send, ao_recv,
          p1_send, p1_recv, p2_send, p2_recv, p3_send, p3_recv,
          wo_sem, wq_sem, x_sem, w_sem):
    my = lax.axis_index("i")

    def dot(a, b):
        return lax.dot_general(a, b, (((a.ndim - 1,), (0,)), ((), ())),
                               preferred_element_type=F32)

    def dot_t(a, b):
        return lax.dot_general(a, b, (((1,), (1,)), ((), ())),
                               preferred_element_type=F32)

    DH = D // 2

    def wo_dma(seq):
        jd = lax.rem(my + seq % N_DEV, N_DEV)
        return pltpu.make_async_copy(
            wo_ref.at[pl.ds(jd * KW, KW), pl.ds((seq // N_DEV) * DH, DH)],
            wo_stage.at[seq % 2],
            wo_sem.at[seq % 2],
        )

    started = []

    wq_dma = pltpu.make_async_copy(
        wq_hbm.at[:, pl.ds(my * KW, KW)], wq_st, wq_sem.at[0])
    wqr_dma = pltpu.make_async_copy(
        wqr_hbm.at[:, pl.ds(my * QW, QW)], wqr_st, wq_sem.at[1])
    def x_dma(t):
        return pltpu.make_async_copy(
            x_hbm.at[pl.ds(t * S, S), :], xst.at[t % 2], x_sem.at[t % 2])

    w_dmas = [
        pltpu.make_async_copy(wdkv_hbm, wdkv_f, w_sem.at[0]),
        pltpu.make_async_copy(wuk_hbm, wuk_f, w_sem.at[1]),
        pltpu.make_async_copy(wuv_hbm, wuv_f, w_sem.at[2]),
    ]
    for dma in w_dmas:
        dma.start()
    wq_dma.start()
    wqr_dma.start()
    wo_dma(0).start()
    wo_dma(1).start()
    x_dma(0).start()
    x_dma(1).start()

    w_dmas[0].wait()
    wdkv_ref[:, :] = wdkv_f[:].astype(BF)
    w_dmas[1].wait()
    wuk_ref[:, :] = wuk_f[:].astype(BF)
    w_dmas[2].wait()
    wuv_ref[:, :] = wuv_f[:].astype(BF)

    for d in range(1, 0 if _DIAG_SKIP_COMM else N_DEV):
        p = lax.rem(my + d, N_DEV)
        for t, src, dstbuf in ((0, wuk_ref, wukc), (1, wuv_ref, wuvc)):
            r = pltpu.make_async_remote_copy(
                src_ref=src.at[:, pl.ds(p * KW, KW)],
                dst_ref=dstbuf.at[pl.ds(my * DCL, DCL), :],
                send_sem=a2a_send.at[t, p],
                recv_sem=a2a_recv.at[t, my],
                device_id=(p,),
                device_id_type=pl.DeviceIdType.MESH,
            )
            r.start()
            started.append(r)

    for t in range(4):
        x_dma(t).wait()
        xb_buf[t * S:(t + 1) * S, :] = xst[t % 2].astype(BF)
        if t + 2 < 4:
            x_dma(t + 2).start()

    xv = xb_buf[:]
    c_all[:, pl.ds(my * DCL, DCL)] = dot(xv, wdkv_ref[:]).astype(BF)
    for d in range(1, 0 if _DIAG_SKIP_COMM else N_DEV):
        p = lax.rem(my + d, N_DEV)
        r = pltpu.make_async_remote_copy(
            src_ref=c_all.at[:, pl.ds(my * DCL, DCL)],
            dst_ref=c_all.at[:, pl.ds(my * DCL, DCL)],
            send_sem=a2a_send.at[2, p],
            recv_sem=a2a_recv.at[2, my],
            device_id=(p,),
            device_id_type=pl.DeviceIdType.MESH,
        )
        r.start()
        started.append(r)

    wukc[pl.ds(my * DCL, DCL), :] = wuk_ref[:, pl.ds(my * KW, KW)]
    wuvc[pl.ds(my * DCL, DCL), :] = wuv_ref[:, pl.ds(my * KW, KW)]

    kr_all = dot(xv, wkr_ref[:]).astype(BF)
    wq_dma.wait()
    q_all = dot(xv, wq_st[:].astype(BF)).astype(BF)
    wqr_dma.wait()
    qr_all = dot(xv, wqr_st[:].astype(BF)).astype(BF)

    for d in range(1, 0 if _DIAG_SKIP_COMM else N_DEV):
        s = lax.rem(my + d, N_DEV)
        for t, dstbuf in ((0, wukc), (1, wuvc)):
            r = pltpu.make_async_remote_copy(
                src_ref=dstbuf.at[pl.ds(s * DCL, DCL), :],
                dst_ref=dstbuf.at[pl.ds(s * DCL, DCL), :],
                send_sem=a2a_send.at[t, s],
                recv_sem=a2a_recv.at[t, s],
                device_id=(s,),
                device_id_type=pl.DeviceIdType.MESH,
            )
            r.wait_recv()
        r = pltpu.make_async_remote_copy(
            src_ref=c_all.at[:, pl.ds(s * DCL, DCL)],
            dst_ref=c_all.at[:, pl.ds(s * DCL, DCL)],
            send_sem=a2a_send.at[2, s],
            recv_sem=a2a_recv.at[2, s],
            device_id=(s,),
            device_id_type=pl.DeviceIdType.MESH,
        )
        r.wait_recv()

    cv = c_all[:]
    k_all = dot(cv, wukc[:]).astype(BF)
    v_all = dot(cv, wuvc[:]).astype(BF)

    for b in range(B):
        r0 = b * S
        krb = kr_all[r0:r0 + S, :]
        for h in range(HL):
            c0 = h * Dh
            q = q_all[r0:r0 + S, c0:c0 + Dh]
            k = k_all[r0:r0 + S, c0:c0 + Dh]
            v = v_all[r0:r0 + S, c0:c0 + Dh]
            qr = qr_all[r0:r0 + S, h * Dr:(h + 1) * Dr]
            sc = (dot_t(q, k) + dot_t(qr, krb)) * SCALE
            mx = jnp.max(sc, axis=1, keepdims=True)
            pr = jnp.exp(sc - mx)
            pr = pr / jnp.sum(pr, axis=1, keepdims=True)
            o = dot(pr.astype(BF), v)
            obuf[r0:r0 + S, c0:c0 + Dh] = o.astype(BF)
        for p in (2 * b, 2 * b + 1) if not _DIAG_SKIP_COMM else ():
            @pl.when(p != my)
            def _send(p=p):
                r = pltpu.make_async_remote_copy(
                    src_ref=obuf.at[pl.ds(p * SL, SL), :],
                    dst_ref=orows.at[pl.ds(my * SL, SL), :],
                    send_sem=ao_send.at[p],
                    recv_sem=ao_recv.at[my],
                    device_id=(p,),
                    device_id_type=pl.DeviceIdType.MESH,
                )
                r.start()

            @pl.when(p == my)
            def _copy(p=p):
                orows[p * SL:(p + 1) * SL, :] = obuf[p * SL:(p + 1) * SL, :]

    nbr = [my ^ 1, my ^ 3, my ^ 4]
    relay_src = [my ^ 3, my ^ 4, my ^ 1]
    need_recv = [1, 2, 0]
    p2_chunk = [my ^ 2, my ^ 7, my ^ 5]

    def ag_rdma(cid, send_sem, recv_sem, dev, h):
        ref = out_ref.at[pl.ds(cid * SL, SL), pl.ds(h * DH, DH)]
        return pltpu.make_async_remote_copy(
            src_ref=ref, dst_ref=ref, send_sem=send_sem, recv_sem=recv_sem,
            device_id=(dev,), device_id_type=pl.DeviceIdType.MESH,
        )

    for h in (0, 1):
        acc = jnp.zeros((SL, DH), F32)
        for d in range(N_DEV):
            jd = lax.rem(my + d, N_DEV)
            if h == 0 and d > 0 and not _DIAG_SKIP_COMM:
                r = pltpu.make_async_remote_copy(
                    src_ref=orows.at[pl.ds(jd * SL, SL), :],
                    dst_ref=orows.at[pl.ds(jd * SL, SL), :],
                    send_sem=ao_send.at[jd],
                    recv_sem=ao_recv.at[jd],
                    device_id=(jd,),
                    device_id_type=pl.DeviceIdType.MESH,
                )
                r.wait_recv()
            seq = h * N_DEV + d
            wo_dma(seq).wait()
            acc = acc + dot(orows[pl.ds(jd * SL, SL), :],
                            wo_stage[seq % 2].astype(BF))
            if seq + 2 < 2 * N_DEV:
                wo_dma(seq + 2).start()
        out_ref[pl.ds(my * SL, SL), pl.ds(h * DH, DH)] = acc.astype(BF)
        if not _DIAG_SKIP_AG:
            for l in range(3):
                r = ag_rdma(my, p1_send.at[l, h], p1_recv.at[l, h], nbr[l], h)
                r.start()
                started.append(r)

    if not _DIAG_SKIP_AG:
        for h in (0, 1):
            for l in range(3):
                j = need_recv[l]
                ag_rdma(nbr[j], p1_send.at[j, h], p1_recv.at[j, h],
                        nbr[j], h).wait_recv()
                r = ag_rdma(relay_src[l], p2_send.at[l, h], p2_recv.at[l, h],
                            nbr[l], h)
                r.start()
                started.append(r)
        for h in (0, 1):
            for l in range(3):
                ag_rdma(p2_chunk[l], p2_send.at[l, h], p2_recv.at[l, h],
                        nbr[l], h).wait_recv()
            src_cid = my ^ (7 if h == 0 else 5)
            r = ag_rdma(src_cid, p3_send.at[h], p3_recv.at[h], nbr[h], h)
            r.start()
            started.append(r)
        for h in (0, 1):
            ag_rdma(my ^ 6, p3_send.at[h], p3_recv.at[h], nbr[h], h).wait_recv()

    for r in started:
        r.wait_send()
    for p in range(0 if _DIAG_SKIP_COMM else N_DEV):
        @pl.when(p != my)
        def _waits(p=p):
            r = pltpu.make_async_remote_copy(
                src_ref=obuf.at[pl.ds(p * SL, SL), :],
                dst_ref=orows.at[pl.ds(my * SL, SL), :],
                send_sem=ao_send.at[p],
                recv_sem=ao_recv.at[my],
                device_id=(p,),
                device_id_type=pl.DeviceIdType.MESH,
            )
            r.wait_send()


def kernel(x, Wdkv, Wuk, Wuv, Wq, Wqr, Wkr, Wo):
    xf = x.reshape(M, D)
    wkr = Wkr.astype(BF)

    out = pl.pallas_call(
        _body,
        out_shape=jax.ShapeDtypeStruct((M, D), BF),
        in_specs=[pl.BlockSpec(memory_space=pl.ANY)] * 4
        + [pl.BlockSpec(memory_space=pltpu.VMEM)]
        + [pl.BlockSpec(memory_space=pl.ANY)] * 3,
        out_specs=pl.BlockSpec(memory_space=pltpu.VMEM),
        scratch_shapes=[
            pltpu.VMEM((M, DC), BF),
            pltpu.VMEM((DC, KW), BF),
            pltpu.VMEM((DC, KW), BF),
            pltpu.VMEM((M, KW), BF),
            pltpu.VMEM((M, KW), BF),
            pltpu.VMEM((2, KW, D // 2), F32),
            pltpu.VMEM((D, KW), F32),
            pltpu.VMEM((D, QW), F32),
            pltpu.VMEM((M, D), BF),
            pltpu.VMEM((2, S, D), F32),
            pltpu.VMEM((D, DCL), F32),
            pltpu.VMEM((DCL, D), F32),
            pltpu.VMEM((DCL, D), F32),
            pltpu.VMEM((D, DCL), BF),
            pltpu.VMEM((DCL, D), BF),
            pltpu.VMEM((DCL, D), BF),
            pltpu.SemaphoreType.DMA((3, N_DEV)),
            pltpu.SemaphoreType.DMA((3, N_DEV)),
            pltpu.SemaphoreType.DMA((N_DEV,)),
            pltpu.SemaphoreType.DMA((N_DEV,)),
            pltpu.SemaphoreType.DMA((3, 2)),
            pltpu.SemaphoreType.DMA((3, 2)),
            pltpu.SemaphoreType.DMA((3, 2)),
            pltpu.SemaphoreType.DMA((3, 2)),
            pltpu.SemaphoreType.DMA((2,)),
            pltpu.SemaphoreType.DMA((2,)),
            pltpu.SemaphoreType.DMA((2,)),
            pltpu.SemaphoreType.DMA((2,)),
            pltpu.SemaphoreType.DMA((2,)),
            pltpu.SemaphoreType.DMA((3,)),
        ],
        compiler_params=pltpu.CompilerParams(
            vmem_limit_bytes=64 * 1024 * 1024,
        ),
    )(xf, Wdkv, Wuk, Wuv, wkr, Wq, Wqr, Wo)
    return out.reshape(B, S, D).astype(jnp.float32)


# device time: 121361 ns/iter; 1.1983x vs baseline; 1.0247x over previous
import jax
import jax.numpy as jnp
from jax import lax
from jax.experimental import pallas as pl
from jax.experimental.pallas import tpu as pltpu

N_DEV = 8
B, S, D = 4, 256, 4096
M = B * S
DC = 1024
DCL = DC // N_DEV
H, Dh, Dr = 32, 128, 64
HL = H // N_DEV
KW = HL * Dh
QW = HL * Dr
SL = M // N_DEV
SCALE = (Dh + Dr) ** -0.5

BF = jnp.bfloat16
F32 = jnp.float32

import os
_DIAG_SKIP_AG = bool(int(os.environ.get("DIAG_SKIP_AG", "0")))
_DIAG_SKIP_COMM = bool(int(os.environ.get("DIAG_SKIP_COMM", "0")))
_DIAG_SKIP_AG = _DIAG_SKIP_AG or _DIAG_SKIP_COMM


def _body(x_hbm, wdkv_hbm, wuk_hbm, wuv_hbm, wkr_ref, wq_hbm, wqr_hbm, wo_ref,
          out_ref,
          c_all, wukc, wuvc, obuf, orows, wo_stage, wq_st, wqr_st, xb_buf, xst,
          wdkv_f, wuk_f, wuv_f, wdkv_ref, wuk_ref, wuv_ref,
          a2a_send, a2a_recv, ao_send, ao_recv,
          p1_send, p1_recv, p2_send, p2_recv, p3_send, p3_recv,
          wo_sem, wq_sem, x_sem, w_sem):
    my = lax.axis_index("i")

    def dot(a, b):
        return lax.dot_general(a, b, (((a.ndim - 1,), (0,)), ((), ())),
                               preferred_element_type=F32)

    def dot_t(a, b):
        return lax.dot_general(a, b, (((1,), (1,)), ((), ())),
                               preferred_element_type=F32)

    DH = D // 2

    def wo_dma(seq):
        jd = lax.rem(my + seq % N_DEV, N_DEV)
        return pltpu.make_async_copy(
            wo_ref.at[pl.ds(jd * KW, KW), pl.ds((seq // N_DEV) * DH, DH)],
            wo_stage.at[seq % 2],
            wo_sem.at[seq % 2],
        )

    started = []

    wq_dma = pltpu.make_async_copy(
        wq_hbm.at[:, pl.ds(my * KW, KW)], wq_st, wq_sem.at[0])
    wqr_dma = pltpu.make_async_copy(
        wqr_hbm.at[:, pl.ds(my * QW, QW)], wqr_st, wq_sem.at[1])
    def x_dma(t):
        return pltpu.make_async_copy(
            x_hbm.at[pl.ds(t * S, S), :], xst.at[t % 2], x_sem.at[t % 2])

    w_dmas = [
        pltpu.make_async_copy(wdkv_hbm, wdkv_f, w_sem.at[0]),
        pltpu.make_async_copy(wuk_hbm, wuk_f, w_sem.at[1]),
        pltpu.make_async_copy(wuv_hbm, wuv_f, w_sem.at[2]),
    ]
    for dma in w_dmas:
        dma.start()
    wq_dma.start()
    wqr_dma.start()
    wo_dma(0).start()
    wo_dma(1).start()
    x_dma(0).start()
    x_dma(1).start()

    w_dmas[0].wait()
    wdkv_ref[:, :] = wdkv_f[:].astype(BF)
    w_dmas[1].wait()
    wuk_ref[:, :] = wuk_f[:].astype(BF)
    w_dmas[2].wait()
    wuv_ref[:, :] = wuv_f[:].astype(BF)

    for d in range(1, 0 if _DIAG_SKIP_COMM else N_DEV):
        p = lax.rem(my + d, N_DEV)
        for t, src, dstbuf in ((0, wuk_ref, wukc), (1, wuv_ref, wuvc)):
            r = pltpu.make_async_remote_copy(
                src_ref=src.at[:, pl.ds(p * KW, KW)],
                dst_ref=dstbuf.at[pl.ds(my * DCL, DCL), :],
                send_sem=a2a_send.at[t, p],
                recv_sem=a2a_recv.at[t, my],
                device_id=(p,),
                device_id_type=pl.DeviceIdType.MESH,
            )
            r.start()
            started.append(r)

    for t in range(4):
        x_dma(t).wait()
        xb_buf[t * S:(t + 1) * S, :] = xst[t % 2].astype(BF)
        if t + 2 < 4:
            x_dma(t + 2).start()

    xv = xb_buf[:]
    c_all[:, pl.ds(my * DCL, DCL)] = dot(xv, wdkv_ref[:]).astype(BF)
    for d in range(1, 0 if _DIAG_SKIP_COMM else N_DEV):
        p = lax.rem(my + d, N_DEV)
        r = pltpu.make_async_remote_copy(
            src_ref=c_all.at[:, pl.ds(my * DCL, DCL)],
            dst_ref=c_all.at[:, pl.ds(my * DCL, DCL)],
            send_sem=a2a_send.at[2, p],
            recv_sem=a2a_recv.at[2, my],
            device_id=(p,),
            device_id_type=pl.DeviceIdType.MESH,
        )
        r.start()
        started.append(r)

    wukc[pl.ds(my * DCL, DCL), :] = wuk_ref[:, pl.ds(my * KW, KW)]
    wuvc[pl.ds(my * DCL, DCL), :] = wuv_ref[:, pl.ds(my * KW, KW)]

    kr_all = dot(xv, wkr_ref[:]).astype(BF)
    wq_dma.wait()
    q_all = dot(xv, wq_st[:].astype(BF)).astype(BF)
    wqr_dma.wait()
    qr_all = dot(xv, wqr_st[:].astype(BF)).astype(BF)

    for d in range(1, 0 if _DIAG_SKIP_COMM else N_DEV):
        s = lax.rem(my + d, N_DEV)
        for t, dstbuf in ((0, wukc), (1, wuvc)):
            r = pltpu.make_async_remote_copy(
                src_ref=dstbuf.at[pl.ds(s * DCL, DCL), :],
                dst_ref=dstbuf.at[pl.ds(s * DCL, DCL), :],
                send_sem=a2a_send.at[t, s],
                recv_sem=a2a_recv.at[t, s],
                device_id=(s,),
                device_id_type=pl.DeviceIdType.MESH,
            )
            r.wait_recv()
        r = pltpu.make_async_remote_copy(
            src_ref=c_all.at[:, pl.ds(s * DCL, DCL)],
            dst_ref=c_all.at[:, pl.ds(s * DCL, DCL)],
            send_sem=a2a_send.at[2, s],
            recv_sem=a2a_recv.at[2, s],
            device_id=(s,),
            device_id_type=pl.DeviceIdType.MESH,
        )
        r.wait_recv()

    cv = c_all[:]
    k_all = dot(cv, wukc[:]).astype(BF)
    v_all = dot(cv, wuvc[:]).astype(BF)

    DHR = Dh + Dr
    qcat = jnp.concatenate(
        [x for h in range(HL)
         for x in (q_all[:, h * Dh:(h + 1) * Dh],
                   qr_all[:, h * Dr:(h + 1) * Dr])], axis=1)
    kcat = jnp.concatenate(
        [x for h in range(HL)
         for x in (k_all[:, h * Dh:(h + 1) * Dh], kr_all)], axis=1)
    for b in range(B):
        r0 = b * S
        for h in range(HL):
            c0 = h * Dh
            v = v_all[r0:r0 + S, c0:c0 + Dh]
            cs = slice(h * DHR, (h + 1) * DHR)
            sc = dot_t(qcat[r0:r0 + S, cs], kcat[r0:r0 + S, cs]) * SCALE
            mx = jnp.max(sc, axis=1, keepdims=True)
            pr = jnp.exp(sc - mx)
            pr = pr / jnp.sum(pr, axis=1, keepdims=True)
            o = dot(pr.astype(BF), v)
            obuf[r0:r0 + S, c0:c0 + Dh] = o.astype(BF)
        for p in (2 * b, 2 * b + 1) if not _DIAG_SKIP_COMM else ():
            @pl.when(p != my)
            def _send(p=p):
                r = pltpu.make_async_remote_copy(
                    src_ref=obuf.at[pl.ds(p * SL, SL), :],
                    dst_ref=orows.at[pl.ds(my * SL, SL), :],
                    send_sem=ao_send.at[p],
                    recv_sem=ao_recv.at[my],
                    device_id=(p,),
                    device_id_type=pl.DeviceIdType.MESH,
                )
                r.start()

            @pl.when(p == my)
            def _copy(p=p):
                orows[p * SL:(p + 1) * SL, :] = obuf[p * SL:(p + 1) * SL, :]

    nbr = [my ^ 1, my ^ 3, my ^ 4]
    relay_src = [my ^ 3, my ^ 4, my ^ 1]
    need_recv = [1, 2, 0]
    p2_chunk = [my ^ 2, my ^ 7, my ^ 5]

    def ag_rdma(cid, send_sem, recv_sem, dev, h):
        ref = out_ref.at[pl.ds(cid * SL, SL), pl.ds(h * DH, DH)]
        return pltpu.make_async_remote_copy(
            src_ref=ref, dst_ref=ref, send_sem=send_sem, recv_sem=recv_sem,
            device_id=(dev,), device_id_type=pl.DeviceIdType.MESH,
        )

    for h in (0, 1):
        acc = jnp.zeros((SL, DH), F32)
        for d in range(N_DEV):
            jd = lax.rem(my + d, N_DEV)
            if h == 0 and d > 0 and not _DIAG_SKIP_COMM:
                r = pltpu.make_async_remote_copy(
                    src_ref=orows.at[pl.ds(jd * SL, SL), :],
                    dst_ref=orows.at[pl.ds(jd * SL, SL), :],
                    send_sem=ao_send.at[jd],
                    recv_sem=ao_recv.at[jd],
                    device_id=(jd,),
                    device_id_type=pl.DeviceIdType.MESH,
                )
                r.wait_recv()
            seq = h * N_DEV + d
            wo_dma(seq).wait()
            acc = acc + dot(orows[pl.ds(jd * SL, SL), :],
                            wo_stage[seq % 2].astype(BF))
            if seq + 2 < 2 * N_DEV:
                wo_dma(seq + 2).start()
        out_ref[pl.ds(my * SL, SL), pl.ds(h * DH, DH)] = acc.astype(BF)
        if not _DIAG_SKIP_AG:
            for l in range(3):
                r = ag_rdma(my, p1_send.at[l, h], p1_recv.at[l, h], nbr[l], h)
                r.start()
                started.append(r)

    if not _DIAG_SKIP_AG:
        for h in (0, 1):
            for l in range(3):
                j = need_recv[l]
                ag_rdma(nbr[j], p1_send.at[j, h], p1_recv.at[j, h],
                        nbr[j], h).wait_recv()
                r = ag_rdma(relay_src[l], p2_send.at[l, h], p2_recv.at[l, h],
                            nbr[l], h)
                r.start()
                started.append(r)
        for h in (0, 1):
            for l in range(3):
                ag_rdma(p2_chunk[l], p2_send.at[l, h], p2_recv.at[l, h],
                        nbr[l], h).wait_recv()
            src_cid = my ^ (7 if h == 0 else 5)
            r = ag_rdma(src_cid, p3_send.at[h], p3_recv.at[h], nbr[h], h)
            r.start()
            started.append(r)
        for h in (0, 1):
            ag_rdma(my ^ 6, p3_send.at[h], p3_recv.at[h], nbr[h], h).wait_recv()

    for r in started:
        r.wait_send()
    for p in range(0 if _DIAG_SKIP_COMM else N_DEV):
        @pl.when(p != my)
        def _waits(p=p):
            r = pltpu.make_async_remote_copy(
                src_ref=obuf.at[pl.ds(p * SL, SL), :],
                dst_ref=orows.at[pl.ds(my * SL, SL), :],
                send_sem=ao_send.at[p],
                recv_sem=ao_recv.at[my],
                device_id=(p,),
                device_id_type=pl.DeviceIdType.MESH,
            )
            r.wait_send()


def kernel(x, Wdkv, Wuk, Wuv, Wq, Wqr, Wkr, Wo):
    xf = x.reshape(M, D)
    wkr = Wkr.astype(BF)

    out = pl.pallas_call(
        _body,
        out_shape=jax.ShapeDtypeStruct((M, D), BF),
        in_specs=[pl.BlockSpec(memory_space=pl.ANY)] * 4
        + [pl.BlockSpec(memory_space=pltpu.VMEM)]
        + [pl.BlockSpec(memory_space=pl.ANY)] * 3,
        out_specs=pl.BlockSpec(memory_space=pltpu.VMEM),
        scratch_shapes=[
            pltpu.VMEM((M, DC), BF),
            pltpu.VMEM((DC, KW), BF),
            pltpu.VMEM((DC, KW), BF),
            pltpu.VMEM((M, KW), BF),
            pltpu.VMEM((M, KW), BF),
            pltpu.VMEM((2, KW, D // 2), F32),
            pltpu.VMEM((D, KW), F32),
            pltpu.VMEM((D, QW), F32),
            pltpu.VMEM((M, D), BF),
            pltpu.VMEM((2, S, D), F32),
            pltpu.VMEM((D, DCL), F32),
            pltpu.VMEM((DCL, D), F32),
            pltpu.VMEM((DCL, D), F32),
            pltpu.VMEM((D, DCL), BF),
            pltpu.VMEM((DCL, D), BF),
            pltpu.VMEM((DCL, D), BF),
            pltpu.SemaphoreType.DMA((3, N_DEV)),
            pltpu.SemaphoreType.DMA((3, N_DEV)),
            pltpu.SemaphoreType.DMA((N_DEV,)),
            pltpu.SemaphoreType.DMA((N_DEV,)),
            pltpu.SemaphoreType.DMA((3, 2)),
            pltpu.SemaphoreType.DMA((3, 2)),
            pltpu.SemaphoreType.DMA((3, 2)),
            pltpu.SemaphoreType.DMA((3, 2)),
            pltpu.SemaphoreType.DMA((2,)),
            pltpu.SemaphoreType.DMA((2,)),
            pltpu.SemaphoreType.DMA((2,)),
            pltpu.SemaphoreType.DMA((2,)),
            pltpu.SemaphoreType.DMA((2,)),
            pltpu.SemaphoreType.DMA((3,)),
        ],
        compiler_params=pltpu.CompilerParams(
            vmem_limit_bytes=64 * 1024 * 1024,
        ),
    )(xf, Wdkv, Wuk, Wuv, wkr, Wq, Wqr, Wo)
    return out.reshape(B, S, D).astype(jnp.float32)


# device time: 120825 ns/iter; 1.2036x vs baseline; 1.0044x over previous
import jax
import jax.numpy as jnp
from jax import lax
from jax.experimental import pallas as pl
from jax.experimental.pallas import tpu as pltpu

N_DEV = 8
B, S, D = 4, 256, 4096
M = B * S
DC = 1024
DCL = DC // N_DEV
H, Dh, Dr = 32, 128, 64
HL = H // N_DEV
KW = HL * Dh
QW = HL * Dr
SL = M // N_DEV
SCALE = (Dh + Dr) ** -0.5

BF = jnp.bfloat16
F32 = jnp.float32

import os
_DIAG_SKIP_AG = bool(int(os.environ.get("DIAG_SKIP_AG", "0")))
_DIAG_SKIP_COMM = bool(int(os.environ.get("DIAG_SKIP_COMM", "0")))
_DIAG_SKIP_AG = _DIAG_SKIP_AG or _DIAG_SKIP_COMM


def _body(x_hbm, wdkv_hbm, wuk_hbm, wuv_hbm, wkr_ref, wq_hbm, wqr_hbm, wo_ref,
          out_ref,
          c_all, wukc, wuvc, obuf, orows, wo_stage, wq_st, wqr_st, xb_buf, xst,
          wdkv_f, wuk_f, wuv_f, wdkv_ref, wuk_ref, wuv_ref,
          a2a_send, a2a_recv, ao_send, ao_recv,
          p1_send, p1_recv, p2_send, p2_recv, p3_send, p3_recv,
          wo_sem, wq_sem, x_sem, w_sem):
    my = lax.axis_index("i")

    def dot(a, b):
        return lax.dot_general(a, b, (((a.ndim - 1,), (0,)), ((), ())),
                               preferred_element_type=F32)

    def dot_t(a, b):
        return lax.dot_general(a, b, (((1,), (1,)), ((), ())),
                               preferred_element_type=F32)

    DH = D // 2

    def wo_dma(seq):
        jd = lax.rem(my + seq % N_DEV, N_DEV)
        return pltpu.make_async_copy(
            wo_ref.at[pl.ds(jd * KW, KW), pl.ds((seq // N_DEV) * DH, DH)],
            wo_stage.at[seq % 2],
            wo_sem.at[seq % 2],
        )

    started = []

    wq_dma = pltpu.make_async_copy(
        wq_hbm.at[:, pl.ds(my * KW, KW)], wq_st, wq_sem.at[0])
    wqr_dma = pltpu.make_async_copy(
        wqr_hbm.at[:, pl.ds(my * QW, QW)], wqr_st, wq_sem.at[1])
    def x_dma(t):
        return pltpu.make_async_copy(
            x_hbm.at[pl.ds(t * S, S), :], xst.at[t % 2], x_sem.at[t % 2])

    w_dmas = [
        pltpu.make_async_copy(wdkv_hbm, wdkv_f, w_sem.at[0]),
        pltpu.make_async_copy(wuk_hbm, wuk_f, w_sem.at[1]),
        pltpu.make_async_copy(wuv_hbm, wuv_f, w_sem.at[2]),
    ]
    for dma in w_dmas:
        dma.start()
    wq_dma.start()
    wqr_dma.start()
    wo_dma(0).start()
    wo_dma(1).start()
    x_dma(0).start()
    x_dma(1).start()

    w_dmas[0].wait()
    wdkv_ref[:, :] = wdkv_f[:].astype(BF)
    w_dmas[1].wait()
    wuk_ref[:, :] = wuk_f[:].astype(BF)
    w_dmas[2].wait()
    wuv_ref[:, :] = wuv_f[:].astype(BF)

    for d in range(1, 0 if _DIAG_SKIP_COMM else N_DEV):
        p = lax.rem(my + d, N_DEV)
        for t, src, dstbuf in ((0, wuk_ref, wukc), (1, wuv_ref, wuvc)):
            r = pltpu.make_async_remote_copy(
                src_ref=src.at[:, pl.ds(p * KW, KW)],
                dst_ref=dstbuf.at[pl.ds(my * DCL, DCL), :],
                send_sem=a2a_send.at[t, p],
                recv_sem=a2a_recv.at[t, my],
                device_id=(p,),
                device_id_type=pl.DeviceIdType.MESH,
            )
            r.start()
            started.append(r)

    for t in range(4):
        x_dma(t).wait()
        xb_buf[t * S:(t + 1) * S, :] = xst[t % 2].astype(BF)
        if t + 2 < 4:
            x_dma(t + 2).start()

    xv = xb_buf[:]
    c_all[:, pl.ds(my * DCL, DCL)] = dot(xv, wdkv_ref[:]).astype(BF)
    for d in range(1, 0 if _DIAG_SKIP_COMM else N_DEV):
        p = lax.rem(my + d, N_DEV)
        r = pltpu.make_async_remote_copy(
            src_ref=c_all.at[:, pl.ds(my * DCL, DCL)],
            dst_ref=c_all.at[:, pl.ds(my * DCL, DCL)],
            send_sem=a2a_send.at[2, p],
            recv_sem=a2a_recv.at[2, my],
            device_id=(p,),
            device_id_type=pl.DeviceIdType.MESH,
        )
        r.start()
        started.append(r)

    wukc[pl.ds(my * DCL, DCL), :] = wuk_ref[:, pl.ds(my * KW, KW)]
    wuvc[pl.ds(my * DCL, DCL), :] = wuv_ref[:, pl.ds(my * KW, KW)]

    kr_all = dot(xv, wkr_ref[:]).astype(BF)
    wq_dma.wait()
    q_all = dot(xv, wq_st[:].astype(BF)).astype(BF)
    wqr_dma.wait()
    qr_all = dot(xv, wqr_st[:].astype(BF)).astype(BF)

    for d in range(1, 0 if _DIAG_SKIP_COMM else N_DEV):
        s = lax.rem(my + d, N_DEV)
        for t, dstbuf in ((0, wukc), (1, wuvc)):
            r = pltpu.make_async_remote_copy(
                src_ref=dstbuf.at[pl.ds(s * DCL, DCL), :],
                dst_ref=dstbuf.at[pl.ds(s * DCL, DCL), :],
                send_sem=a2a_send.at[t, s],
                recv_sem=a2a_recv.at[t, s],
                device_id=(s,),
                device_id_type=pl.DeviceIdType.MESH,
            )
            r.wait_recv()
        r = pltpu.make_async_remote_copy(
            src_ref=c_all.at[:, pl.ds(s * DCL, DCL)],
            dst_ref=c_all.at[:, pl.ds(s * DCL, DCL)],
            send_sem=a2a_send.at[2, s],
            recv_sem=a2a_recv.at[2, s],
            device_id=(s,),
            device_id_type=pl.DeviceIdType.MESH,
        )
        r.wait_recv()

    cv = c_all[:]
    k_all = dot(cv, wukc[:]).astype(BF)
    v_all = dot(cv, wuvc[:]).astype(BF)

    DHR = Dh + Dr
    qcat = jnp.concatenate(
        [x for h in range(HL)
         for x in (q_all[:, h * Dh:(h + 1) * Dh],
                   qr_all[:, h * Dr:(h + 1) * Dr])], axis=1)
    kcat = jnp.concatenate(
        [x for h in range(HL)
         for x in (k_all[:, h * Dh:(h + 1) * Dh], kr_all)], axis=1)
    for b in range(B):
        r0 = b * S
        for h in range(HL):
            c0 = h * Dh
            v = v_all[r0:r0 + S, c0:c0 + Dh]
            cs = slice(h * DHR, (h + 1) * DHR)
            sc = dot_t(qcat[r0:r0 + S, cs], kcat[r0:r0 + S, cs]) * SCALE
            pr = jnp.exp(sc)
            o = dot(pr.astype(BF), v) / jnp.sum(pr, axis=1, keepdims=True)
            obuf[r0:r0 + S, c0:c0 + Dh] = o.astype(BF)
        for p in (2 * b, 2 * b + 1) if not _DIAG_SKIP_COMM else ():
            @pl.when(p != my)
            def _send(p=p):
                r = pltpu.make_async_remote_copy(
                    src_ref=obuf.at[pl.ds(p * SL, SL), :],
                    dst_ref=orows.at[pl.ds(my * SL, SL), :],
                    send_sem=ao_send.at[p],
                    recv_sem=ao_recv.at[my],
                    device_id=(p,),
                    device_id_type=pl.DeviceIdType.MESH,
                )
                r.start()

            @pl.when(p == my)
            def _copy(p=p):
                orows[p * SL:(p + 1) * SL, :] = obuf[p * SL:(p + 1) * SL, :]

    nbr = [my ^ 1, my ^ 3, my ^ 4]
    relay_src = [my ^ 3, my ^ 4, my ^ 1]
    need_recv = [1, 2, 0]
    p2_chunk = [my ^ 2, my ^ 7, my ^ 5]

    def ag_rdma(cid, send_sem, recv_sem, dev, h):
        ref = out_ref.at[pl.ds(cid * SL, SL), pl.ds(h * DH, DH)]
        return pltpu.make_async_remote_copy(
            src_ref=ref, dst_ref=ref, send_sem=send_sem, recv_sem=recv_sem,
            device_id=(dev,), device_id_type=pl.DeviceIdType.MESH,
        )

    for h in (0, 1):
        acc = jnp.zeros((SL, DH), F32)
        for d in range(N_DEV):
            jd = lax.rem(my + d, N_DEV)
            if h == 0 and d > 0 and not _DIAG_SKIP_COMM:
                r = pltpu.make_async_remote_copy(
                    src_ref=orows.at[pl.ds(jd * SL, SL), :],
                    dst_ref=orows.at[pl.ds(jd * SL, SL), :],
                    send_sem=ao_send.at[jd],
                    recv_sem=ao_recv.at[jd],
                    device_id=(jd,),
                    device_id_type=pl.DeviceIdType.MESH,
                )
                r.wait_recv()
            seq = h * N_DEV + d
            wo_dma(seq).wait()
            acc = acc + dot(orows[pl.ds(jd * SL, SL), :],
                            wo_stage[seq % 2].astype(BF))
            if seq + 2 < 2 * N_DEV:
                wo_dma(seq + 2).start()
        out_ref[pl.ds(my * SL, SL), pl.ds(h * DH, DH)] = acc.astype(BF)
        if not _DIAG_SKIP_AG:
            for l in range(3):
                r = ag_rdma(my, p1_send.at[l, h], p1_recv.at[l, h], nbr[l], h)
                r.start()
                started.append(r)

    if not _DIAG_SKIP_AG:
        for h in (0, 1):
            for l in range(3):
                j = need_recv[l]
                ag_rdma(nbr[j], p1_send.at[j, h], p1_recv.at[j, h],
                        nbr[j], h).wait_recv()
                r = ag_rdma(relay_src[l], p2_send.at[l, h], p2_recv.at[l, h],
                            nbr[l], h)
                r.start()
                started.append(r)
        for h in (0, 1):
            for l in range(3):
                ag_rdma(p2_chunk[l], p2_send.at[l, h], p2_recv.at[l, h],
                        nbr[l], h).wait_recv()
            src_cid = my ^ (7 if h == 0 else 5)
            r = ag_rdma(src_cid, p3_send.at[h], p3_recv.at[h], nbr[h], h)
            r.start()
            started.append(r)
        for h in (0, 1):
            ag_rdma(my ^ 6, p3_send.at[h], p3_recv.at[h], nbr[h], h).wait_recv()

    for r in started:
        r.wait_send()
    for p in range(0 if _DIAG_SKIP_COMM else N_DEV):
        @pl.when(p != my)
        def _waits(p=p):
            r = pltpu.make_async_remote_copy(
                src_ref=obuf.at[pl.ds(p * SL, SL), :],
                dst_ref=orows.at[pl.ds(my * SL, SL), :],
                send_sem=ao_send.at[p],
                recv_sem=ao_recv.at[my],
                device_id=(p,),
                device_id_type=pl.DeviceIdType.MESH,
            )
            r.wait_send()


def kernel(x, Wdkv, Wuk, Wuv, Wq, Wqr, Wkr, Wo):
    xf = x.reshape(M, D)
    wkr = Wkr.astype(BF)

    out = pl.pallas_call(
        _body,
        out_shape=jax.ShapeDtypeStruct((M, D), BF),
        in_specs=[pl.BlockSpec(memory_space=pl.ANY)] * 4
        + [pl.BlockSpec(memory_space=pltpu.VMEM)]
        + [pl.BlockSpec(memory_space=pl.ANY)] * 3,
        out_specs=pl.BlockSpec(memory_space=pltpu.VMEM),
        scratch_shapes=[
            pltpu.VMEM((M, DC), BF),
            pltpu.VMEM((DC, KW), BF),
            pltpu.VMEM((DC, KW), BF),
            pltpu.VMEM((M, KW), BF),
            pltpu.VMEM((M, KW), BF),
            pltpu.VMEM((2, KW, D // 2), F32),
            pltpu.VMEM((D, KW), F32),
            pltpu.VMEM((D, QW), F32),
            pltpu.VMEM((M, D), BF),
            pltpu.VMEM((2, S, D), F32),
            pltpu.VMEM((D, DCL), F32),
            pltpu.VMEM((DCL, D), F32),
            pltpu.VMEM((DCL, D), F32),
            pltpu.VMEM((D, DCL), BF),
            pltpu.VMEM((DCL, D), BF),
            pltpu.VMEM((DCL, D), BF),
            pltpu.SemaphoreType.DMA((3, N_DEV)),
            pltpu.SemaphoreType.DMA((3, N_DEV)),
            pltpu.SemaphoreType.DMA((N_DEV,)),
            pltpu.SemaphoreType.DMA((N_DEV,)),
            pltpu.SemaphoreType.DMA((3, 2)),
            pltpu.SemaphoreType.DMA((3, 2)),
            pltpu.SemaphoreType.DMA((3, 2)),
            pltpu.SemaphoreType.DMA((3, 2)),
            pltpu.SemaphoreType.DMA((2,)),
            pltpu.SemaphoreType.DMA((2,)),
            pltpu.SemaphoreType.DMA((2,)),
            pltpu.SemaphoreType.DMA((2,)),
            pltpu.SemaphoreType.DMA((2,)),
            pltpu.SemaphoreType.DMA((3,)),
        ],
        compiler_params=pltpu.CompilerParams(
            vmem_limit_bytes=64 * 1024 * 1024,
        ),
    )(xf, Wdkv, Wuk, Wuv, wkr, Wq, Wqr, Wo)
    return out.reshape(B, S, D).astype(jnp.float32)
